# baseline (speedup 1.0000x reference)
import functools

import jax
import jax.numpy as jnp
from jax import lax
from jax.experimental import pallas as pl
from jax.experimental.pallas import tpu as pltpu
from jax.experimental.pallas import tpu_sc as plsc

NC, NS = 2, 16
NW = NC * NS
CHUNK = 128
NBUF = 3
NCB = 3


def _make_pack_table(V, D):
    rows = 4000
    half = V // 2
    grid = half // rows

    @functools.partial(
        pl.pallas_call,
        grid=(grid,),
        in_specs=[
            pl.BlockSpec((rows, D), lambda g: (g, 0)),
            pl.BlockSpec((rows, D), lambda g: (g + grid, 0)),
        ],
        out_specs=pl.BlockSpec((rows, 2 * D), lambda g: (g, 0)),
        out_shape=jax.ShapeDtypeStruct((half, 2 * D), jnp.float32),
    )
    def run(lo_ref, hi_ref, out_ref):
        out_ref[...] = jnp.concatenate([lo_ref[...], hi_ref[...]], axis=1)

    return run


def _make_gather(n_chunks, chunks_per_w, D):
    mesh = plsc.VectorSubcoreMesh(core_axis_name="c", subcore_axis_name="s")

    @functools.partial(
        pl.kernel,
        out_type=jax.ShapeDtypeStruct((n_chunks, D, CHUNK), jnp.float32),
        mesh=mesh,
        scratch_types=[
            pltpu.VMEM((chunks_per_w, CHUNK), jnp.int32),
            pltpu.VMEM((chunks_per_w, CHUNK), jnp.int32),
            pltpu.VMEM((NBUF, CHUNK, 2 * D), jnp.float32),
            pltpu.VMEM((NCB, D, CHUNK), jnp.float32),
            pltpu.SemaphoreType.DMA,
            pltpu.SemaphoreType.DMA,
        ],
        compiler_params=pltpu.CompilerParams(
            use_tc_tiling_on_sc=False, needs_layout_passes=False),
    )
    def run(gidx_hbm, par_hbm, tbl_hbm, out_hbm, gidx_v, par_v, rows_v,
            comp_v, gsem, osem):
        wid = lax.axis_index("s") * NC + lax.axis_index("c")
        row0 = wid * chunks_per_w
        pltpu.sync_copy(gidx_hbm.at[pl.ds(row0, chunks_per_w)], gidx_v)
        pltpu.sync_copy(par_hbm.at[pl.ds(row0, chunks_per_w)], par_v)

        def fire_gather(c):
            pltpu.async_copy(
                tbl_hbm.at[gidx_v.at[c]], rows_v.at[c % NBUF], gsem)

        def fire_wb(c):
            pltpu.async_copy(
                comp_v.at[c % NCB], out_hbm.at[row0 + c], osem)

        def drain(sem, rows):
            pltpu.make_async_copy(
                tbl_hbm.at[pl.ds(0, rows)],
                rows_v.at[0].at[pl.ds(0, rows)], sem).wait()

        def compact(c):
            buf = rows_v.at[c % NBUF]
            dst = comp_v.at[c % NCB]
            lanes = lax.iota(jnp.int32, 16)
            for rg in range(CHUNK // 16):
                par16 = par_v[c, pl.ds(rg * 16, 16)]
                srow = rg * 16 + lanes
                for d in range(D):
                    vals = plsc.load_gather(buf, [srow, par16 + d])
                    plsc.store_scatter(dst, [lanes * 0 + d, srow], vals)

        for c in range(NBUF - 1):
            fire_gather(c)

        def body(c, _):
            drain(gsem, CHUNK)
            compact(c)
            @pl.when(c + NBUF - 1 < chunks_per_w)
            def _():
                fire_gather(c + NBUF - 1)
            @pl.when(c >= NCB - 1)
            def _():
                drain(osem, D)
            fire_wb(c)
            return ()

        lax.fori_loop(0, chunks_per_w, body, ())
        drain(osem, D)
        drain(osem, D)

    return run


def _make_unflatten(F, B, D, n_chunks):
    per_f = B // CHUNK

    @functools.partial(
        pl.pallas_call,
        grid=(F, per_f),
        in_specs=[pl.BlockSpec((1, D, CHUNK), lambda f, j: (f * per_f + j, 0, 0))],
        out_specs=pl.BlockSpec((1, D, CHUNK), lambda f, j: (f, 0, j)),
        out_shape=jax.ShapeDtypeStruct((F, D, B), jnp.float32),
    )
    def run(in_ref, out_ref):
        out_ref[...] = in_ref[...]

    return run


def kernel(cat_idx, table):
    B, F = cat_idx.shape
    V, D = table.shape
    total = B * F
    n_chunks = total // CHUNK
    chunks_per_w = n_chunks // NW
    half = V // 2

    idx2d = cat_idx.T.reshape(n_chunks, CHUNK).astype(jnp.int32)
    in_lo = idx2d < half
    gidx2d = jnp.where(in_lo, idx2d, idx2d - half)
    par2d = jnp.where(in_lo, 0, D).astype(jnp.int32)

    table2 = _make_pack_table(V, D)(table, table)
    planes = _make_gather(n_chunks, chunks_per_w, D)(gidx2d, par2d, table2)
    fplanes = _make_unflatten(F, B, D, n_chunks)(planes)
    return jnp.transpose(fplanes, (2, 0, 1))


# speedup vs baseline: 2.1016x; 2.1016x over previous
import functools

import jax
import jax.numpy as jnp
from jax import lax
from jax.experimental import pallas as pl
from jax.experimental.pallas import tpu as pltpu
from jax.experimental.pallas import tpu_sc as plsc

NC, NS = 2, 16
NW = NC * NS
CHUNK = 128
NBUF = 4
NCB = 3


def _make_pack_table(V, D):
    rows = 4000
    half = V // 2
    grid = half // rows

    @functools.partial(
        pl.pallas_call,
        grid=(grid,),
        in_specs=[
            pl.BlockSpec((rows, D), lambda g: (g, 0)),
            pl.BlockSpec((rows, D), lambda g: (g + grid, 0)),
        ],
        out_specs=pl.BlockSpec((rows, 2 * D), lambda g: (g, 0)),
        out_shape=jax.ShapeDtypeStruct((half, 2 * D), jnp.float32),
    )
    def run(lo_ref, hi_ref, out_ref):
        out_ref[...] = jnp.concatenate([lo_ref[...], hi_ref[...]], axis=1)

    return run


def _make_gather(n_chunks, chunks_per_w, D):
    mesh = plsc.VectorSubcoreMesh(core_axis_name="c", subcore_axis_name="s")

    @functools.partial(
        pl.kernel,
        out_type=jax.ShapeDtypeStruct((n_chunks, D, CHUNK), jnp.float32),
        mesh=mesh,
        scratch_types=[
            pltpu.VMEM((chunks_per_w, CHUNK), jnp.int32),
            pltpu.VMEM((chunks_per_w, CHUNK), jnp.int32),
            pltpu.VMEM((NBUF, CHUNK, 2 * D), jnp.float32),
            pltpu.VMEM((NCB, D, CHUNK), jnp.float32),
            pltpu.SemaphoreType.DMA,
            pltpu.SemaphoreType.DMA,
        ],
        compiler_params=pltpu.CompilerParams(
            use_tc_tiling_on_sc=False, needs_layout_passes=False,
            disable_bounds_checks=True),
    )
    def run(gidx_hbm, par_hbm, tbl_hbm, out_hbm, gidx_v, par_v, rows_v,
            comp_v, gsem, osem):
        wid = lax.axis_index("s") * NC + lax.axis_index("c")
        row0 = wid * chunks_per_w
        pltpu.sync_copy(gidx_hbm.at[pl.ds(row0, chunks_per_w)], gidx_v)
        pltpu.sync_copy(par_hbm.at[pl.ds(row0, chunks_per_w)], par_v)

        def fire_gather(c):
            pltpu.async_copy(
                tbl_hbm.at[gidx_v.at[c]], rows_v.at[c % NBUF], gsem)

        def fire_wb(c):
            pltpu.async_copy(
                comp_v.at[c % NCB], out_hbm.at[row0 + c], osem)

        def drain(sem, rows):
            pltpu.make_async_copy(
                tbl_hbm.at[pl.ds(0, rows)],
                rows_v.at[0].at[pl.ds(0, rows)], sem).wait()

        def compact(c):
            buf = rows_v.at[c % NBUF]
            dst = comp_v.at[c % NCB]
            lanes = lax.iota(jnp.int32, 16)

            def rg_body(rg, _):
                par16 = par_v[c, pl.ds(rg * 16, 16)]
                srow = rg * 16 + lanes

                def d_body(dq, _):
                    for dd in range(16):
                        d = dq * 16 + dd
                        vals = plsc.load_gather(buf, [srow, par16 + d])
                        plsc.store_scatter(dst, [lanes * 0 + d, srow], vals)
                    return ()

                lax.fori_loop(0, D // 16, d_body, ())
                return ()

            lax.fori_loop(0, CHUNK // 16, rg_body, ())

        for c in range(NBUF - 1):
            fire_gather(c)

        def body(c, _):
            drain(gsem, CHUNK)
            compact(c)
            @pl.when(c + NBUF - 1 < chunks_per_w)
            def _():
                fire_gather(c + NBUF - 1)
            @pl.when(c >= NCB - 1)
            def _():
                drain(osem, D)
            fire_wb(c)
            return ()

        lax.fori_loop(0, chunks_per_w, body, ())
        drain(osem, D)
        drain(osem, D)

    return run


def _make_unflatten(F, B, D, n_chunks):
    k = 32
    per_f = B // CHUNK // k

    @functools.partial(
        pl.pallas_call,
        grid=(F, per_f),
        in_specs=[pl.BlockSpec((k, D, CHUNK), lambda f, j: (f * per_f + j, 0, 0))],
        out_specs=pl.BlockSpec((1, D, k * CHUNK), lambda f, j: (f, 0, j)),
        out_shape=jax.ShapeDtypeStruct((F, D, B), jnp.float32),
    )
    def run(in_ref, out_ref):
        for c in range(k):
            out_ref[0, :, pl.ds(c * CHUNK, CHUNK)] = in_ref[c]

    return run


def kernel(cat_idx, table):
    B, F = cat_idx.shape
    V, D = table.shape
    total = B * F
    n_chunks = total // CHUNK
    chunks_per_w = n_chunks // NW
    half = V // 2

    idx2d = cat_idx.T.reshape(n_chunks, CHUNK).astype(jnp.int32)
    in_lo = idx2d < half
    gidx2d = jnp.where(in_lo, idx2d, idx2d - half)
    par2d = jnp.where(in_lo, 0, D).astype(jnp.int32)

    table2 = _make_pack_table(V, D)(table, table)
    planes = _make_gather(n_chunks, chunks_per_w, D)(gidx2d, par2d, table2)
    fplanes = _make_unflatten(F, B, D, n_chunks)(planes)
    return jnp.transpose(fplanes, (2, 0, 1))


# speedup vs baseline: 2.1040x; 1.0011x over previous
import functools

import jax
import jax.numpy as jnp
from jax import lax
from jax.experimental import pallas as pl
from jax.experimental.pallas import tpu as pltpu
from jax.experimental.pallas import tpu_sc as plsc

NC, NS = 2, 16
NW = NC * NS
CHUNK = 128
NBUF = 4
NCB = 3


def _make_pack_table(V, D):
    rows = 4000
    half = V // 2
    grid = half // rows

    @functools.partial(
        pl.pallas_call,
        grid=(grid,),
        in_specs=[
            pl.BlockSpec((rows, D), lambda g: (g, 0)),
            pl.BlockSpec((rows, D), lambda g: (g + grid, 0)),
        ],
        out_specs=pl.BlockSpec((rows, 2 * D), lambda g: (g, 0)),
        out_shape=jax.ShapeDtypeStruct((half, 2 * D), jnp.float32),
    )
    def run(lo_ref, hi_ref, out_ref):
        out_ref[...] = jnp.concatenate([lo_ref[...], hi_ref[...]], axis=1)

    return run


def _make_gather(n_chunks, chunks_per_w, D):
    mesh = plsc.VectorSubcoreMesh(core_axis_name="c", subcore_axis_name="s")

    @functools.partial(
        pl.kernel,
        out_type=jax.ShapeDtypeStruct((n_chunks, D, CHUNK), jnp.float32),
        mesh=mesh,
        scratch_types=[
            pltpu.VMEM((chunks_per_w, CHUNK), jnp.int32),
            pltpu.VMEM((chunks_per_w, CHUNK), jnp.int32),
            pltpu.VMEM((NBUF, CHUNK, 2 * D), jnp.float32),
            pltpu.VMEM((NCB, D, CHUNK), jnp.float32),
            pltpu.SemaphoreType.DMA,
            pltpu.SemaphoreType.DMA,
        ],
        compiler_params=pltpu.CompilerParams(
            use_tc_tiling_on_sc=False, needs_layout_passes=False,
            disable_bounds_checks=True),
    )
    def run(gidx_hbm, par_hbm, tbl_hbm, out_hbm, gidx_v, par_v, rows_v,
            comp_v, gsem, osem):
        wid = lax.axis_index("s") * NC + lax.axis_index("c")
        row0 = wid * chunks_per_w
        pltpu.sync_copy(gidx_hbm.at[pl.ds(row0, chunks_per_w)], gidx_v)
        pltpu.sync_copy(par_hbm.at[pl.ds(row0, chunks_per_w)], par_v)

        def fire_gather(c):
            pltpu.async_copy(
                tbl_hbm.at[gidx_v.at[c]], rows_v.at[c % NBUF], gsem)

        def fire_wb(c):
            pltpu.async_copy(
                comp_v.at[c % NCB], out_hbm.at[row0 + c], osem)

        def drain(sem, rows):
            pltpu.make_async_copy(
                tbl_hbm.at[pl.ds(0, rows)],
                rows_v.at[0].at[pl.ds(0, rows)], sem).wait()

        def compact(c):
            buf = rows_v.at[c % NBUF]
            dst = comp_v.at[c % NCB]
            lanes = lax.iota(jnp.int32, 16)

            zero = lanes * 0
            for rg in range(CHUNK // 16):
                par16 = par_v[c, pl.ds(rg * 16, 16)]
                srow = rg * 16 + lanes

                def d_body(dq, _, par16=par16, srow=srow):
                    col = par16 + dq * 16
                    dvec = zero + dq * 16
                    for dd in range(16):
                        vals = plsc.load_gather(buf, [srow, col + dd])
                        plsc.store_scatter(dst, [dvec + dd, srow], vals)
                    return ()

                lax.fori_loop(0, D // 16, d_body, ())

        for c in range(NBUF - 1):
            fire_gather(c)

        def body(c, _):
            drain(gsem, CHUNK)
            compact(c)
            @pl.when(c + NBUF - 1 < chunks_per_w)
            def _():
                fire_gather(c + NBUF - 1)
            @pl.when(c >= NCB - 1)
            def _():
                drain(osem, D)
            fire_wb(c)
            return ()

        lax.fori_loop(0, chunks_per_w, body, ())
        drain(osem, D)
        drain(osem, D)

    return run


def _make_unflatten(F, B, D, n_chunks):
    k = 32
    per_f = B // CHUNK // k

    @functools.partial(
        pl.pallas_call,
        grid=(F, per_f),
        in_specs=[pl.BlockSpec((k, D, CHUNK), lambda f, j: (f * per_f + j, 0, 0))],
        out_specs=pl.BlockSpec((1, D, k * CHUNK), lambda f, j: (f, 0, j)),
        out_shape=jax.ShapeDtypeStruct((F, D, B), jnp.float32),
    )
    def run(in_ref, out_ref):
        for c in range(k):
            out_ref[0, :, pl.ds(c * CHUNK, CHUNK)] = in_ref[c]

    return run


def kernel(cat_idx, table):
    B, F = cat_idx.shape
    V, D = table.shape
    total = B * F
    n_chunks = total // CHUNK
    chunks_per_w = n_chunks // NW
    half = V // 2

    idx2d = cat_idx.T.reshape(n_chunks, CHUNK).astype(jnp.int32)
    in_lo = idx2d < half
    gidx2d = jnp.where(in_lo, idx2d, idx2d - half)
    par2d = jnp.where(in_lo, 0, D).astype(jnp.int32)

    table2 = _make_pack_table(V, D)(table, table)
    planes = _make_gather(n_chunks, chunks_per_w, D)(gidx2d, par2d, table2)
    fplanes = _make_unflatten(F, B, D, n_chunks)(planes)
    return jnp.transpose(fplanes, (2, 0, 1))


# speedup vs baseline: 2.9781x; 1.4154x over previous
import functools

import jax
import jax.numpy as jnp
from jax import lax
from jax.experimental import pallas as pl
from jax.experimental.pallas import tpu as pltpu
from jax.experimental.pallas import tpu_sc as plsc

NC, NS = 2, 16
NW = NC * NS
CHUNK = 128


NBUF = 8


def _make_gather(n_chunks, chunks_per_w, D):
    mesh = plsc.VectorSubcoreMesh(core_axis_name="c", subcore_axis_name="s")

    @functools.partial(
        pl.kernel,
        out_type=jax.ShapeDtypeStruct((n_chunks * CHUNK, D), jnp.float32),
        mesh=mesh,
        scratch_types=[
            pltpu.VMEM((chunks_per_w, CHUNK), jnp.int32),
            pltpu.VMEM((NBUF, CHUNK, D), jnp.float32),
            pltpu.SemaphoreType.DMA,
            pltpu.SemaphoreType.DMA,
        ],
        compiler_params=pltpu.CompilerParams(use_tc_tiling_on_sc=False),
    )
    def run(idx_hbm, table_hbm, out_hbm, idx_v, rows_v, gsem, osem):
        wid = lax.axis_index("s") * NC + lax.axis_index("c")
        row0 = wid * chunks_per_w
        pltpu.sync_copy(idx_hbm.at[pl.ds(row0, chunks_per_w)], idx_v)

        def fire_gather(c):
            pltpu.async_copy(
                table_hbm.at[idx_v.at[c]], rows_v.at[c % NBUF], gsem)

        def fire_wb(c):
            pltpu.async_copy(
                rows_v.at[c % NBUF],
                out_hbm.at[pl.ds((row0 + c) * CHUNK, CHUNK)], osem)

        def drain(sem):
            pltpu.make_async_copy(
                out_hbm.at[pl.ds(0, CHUNK)], rows_v.at[0], sem).wait()

        for c in range(NBUF - 2):
            fire_gather(c)

        def body(c, _):
            drain(gsem)
            fire_wb(c)
            @pl.when(c >= 2)
            def _():
                drain(osem)
            @pl.when(c + NBUF - 2 < chunks_per_w)
            def _():
                fire_gather(c + NBUF - 2)
            return ()

        lax.fori_loop(0, chunks_per_w, body, ())
        drain(osem)
        drain(osem)

    return run


def kernel(cat_idx, table):
    B, F = cat_idx.shape
    V, D = table.shape
    total = B * F
    n_chunks = total // CHUNK
    chunks_per_w = n_chunks // NW
    idx2d = cat_idx.reshape(n_chunks, CHUNK).astype(jnp.int32)
    out = _make_gather(n_chunks, chunks_per_w, D)(idx2d, table)
    return out.reshape(B, F, D)
